# baseline (speedup 1.0000x reference)
import functools

import jax
import jax.numpy as jnp
from jax import lax
from jax.experimental import pallas as pl
from jax.experimental.pallas import tpu as pltpu
from jax.experimental.pallas import tpu_sc as plsc

_K = 100
_CAP = 128
_SCALE_F = 4.0
_THRESH_F = 0.01
_CB = 16
_B, _C, _H, _W = 8, 80, 256, 256
_ROWS = _C * _H
_NGRP = _C
_BIGI_VAL = 2 ** 30
_CAPR = 112


def _tc_common(x):
    ninf_row = jnp.full((_CB, 1, _W), -jnp.inf, jnp.float32)
    up = jnp.concatenate([ninf_row, x[:, :-1, :]], axis=1)
    dn = jnp.concatenate([x[:, 1:, :], ninf_row], axis=1)
    v = jnp.maximum(jnp.maximum(up, x), dn)
    ninf_col = jnp.full((_CB, _H, 1), -jnp.inf, jnp.float32)
    lf = jnp.concatenate([ninf_col, v[:, :, :-1]], axis=2)
    rt = jnp.concatenate([v[:, :, 1:], ninf_col], axis=2)
    pooled = jnp.maximum(jnp.maximum(lf, v), rt)
    return jnp.where(pooled == x, x, 0.0)


def _tc_body(hm_ref, sup_ref, rmax_ref):
    sup = _tc_common(hm_ref[0])
    sup_ref[0] = sup
    rmax_ref[0] = jnp.max(sup, axis=-1)


def _tc_body_nosup(hm_ref, rmax_ref):
    sup = _tc_common(hm_ref[0])
    rmax_ref[0] = jnp.max(sup, axis=-1)


_stage1 = pl.pallas_call(
    _tc_body,
    grid=(_B, _C // _CB),
    in_specs=[pl.BlockSpec((1, _CB, _H, _W), lambda b, c: (b, c, 0, 0))],
    out_specs=[
        pl.BlockSpec((1, _CB, _H, _W), lambda b, c: (b, c, 0, 0)),
        pl.BlockSpec((1, _CB, _H), lambda b, c: (b, c, 0)),
    ],
    out_shape=[
        jax.ShapeDtypeStruct((_B, _C, _H, _W), jnp.float32),
        jax.ShapeDtypeStruct((_B, _C, _H), jnp.float32),
    ],
)

_stage1_nosup = pl.pallas_call(
    _tc_body_nosup,
    grid=(_B, _C // _CB),
    in_specs=[pl.BlockSpec((1, _CB, _H, _W), lambda b, c: (b, c, 0, 0))],
    out_specs=[pl.BlockSpec((1, _CB, _H), lambda b, c: (b, c, 0))],
    out_shape=[jax.ShapeDtypeStruct((_B, _C, _H), jnp.float32)],
)


def _sc_body(rmax_hbm, sup_hbm, off_hbm, wh_hbm,
             ids_hbm, sc_hbm, bb_hbm,
             rmax_v, cmax_v, gmax_v, cval_v, crow_v, rows_v, rowsu_v,
             rowsd_v, ccol_v, idx_v, idx2_v, idxr_v, idxu_v, idxd_v,
             upok_v, dnok_v,
             wrk_v,
             xf_v, yf_v, idf_v, sp_v, scv_v, ga_v, gb_v, gc_v, gd_v, bb_v,
             sem):
    wid = lax.axis_index("s") * 2 + lax.axis_index("c")

    @pl.when(wid < _B)
    def _():
        b = wid
        iot = lax.iota(jnp.int32, 16)
        lane0 = iot == 0
        ninf16 = jnp.full((16,), -jnp.inf, jnp.float32)
        _BIGI = jnp.int32(_BIGI_VAL)

        def spl_i(s):
            return jnp.full((16,), s, jnp.int32)

        def spl_f(s):
            return jnp.full((16,), s, jnp.float32)

        pltpu.sync_copy(rmax_hbm.at[b], rmax_v)

        @plsc.parallel_loop(0, _ROWS // 16, unroll=8)
        def _cpre(c):
            m = jnp.max(plsc.load_gather(rmax_v, [c * 16 + iot]))
            plsc.store_scatter(cmax_v, [spl_i(c)], spl_f(m), mask=lane0)

        @plsc.parallel_loop(0, _NGRP, unroll=8)
        def _gpre(g):
            m = jnp.max(plsc.load_gather(cmax_v, [g * 16 + iot]))
            plsc.store_scatter(gmax_v, [spl_i(g)], spl_f(m), mask=lane0)

        for t in range(_CAP // 16):
            cval_v[pl.ds(t * 16, 16)] = ninf16
            crow_v[pl.ds(t * 16, 16)] = jnp.zeros((16,), jnp.int32)

        def row_step(k, carry):
            def sm(t, acc):
                return jnp.maximum(acc, gmax_v[pl.ds(t * 16, 16)])
            m = jnp.max(lax.fori_loop(0, _NGRP // 16, sm, ninf16,
                                      unroll=True))

            def fg(t, acc):
                v = gmax_v[pl.ds(t * 16, 16)]
                return jnp.minimum(acc, jnp.where(v == m, iot + t * 16, _BIGI))
            g = jnp.min(lax.fori_loop(0, _NGRP // 16, fg,
                                      jnp.full((16,), _BIGI_VAL, jnp.int32),
                                      unroll=True))

            cs = plsc.load_gather(cmax_v, [g * 16 + iot])
            c = g * 16 + jnp.min(jnp.where(cs == m, iot, _BIGI))

            rs = plsc.load_gather(rmax_v, [c * 16 + iot])
            r = c * 16 + jnp.min(jnp.where(rs == m, iot, _BIGI))

            plsc.store_scatter(cval_v, [spl_i(k)], spl_f(m), mask=lane0)
            plsc.store_scatter(crow_v, [spl_i(k)], spl_i(r), mask=lane0)
            plsc.store_scatter(rmax_v, [spl_i(r)], spl_f(-1.0), mask=lane0)

            nc = jnp.max(plsc.load_gather(rmax_v, [c * 16 + iot]))
            plsc.store_scatter(cmax_v, [spl_i(c)], spl_f(nc), mask=lane0)
            ng = jnp.max(plsc.load_gather(cmax_v, [g * 16 + iot]))
            plsc.store_scatter(gmax_v, [spl_i(g)], spl_f(ng), mask=lane0)
            return carry
        lax.fori_loop(0, _K, row_step, 0)

        for t in range(_CAPR // 16):
            s = pl.ds(t * 16, 16)
            rw = crow_v[s]
            y = rw - (rw // 256) * 256
            upok_v[s] = jnp.where(y > 0, 1.0, 0.0)
            dnok_v[s] = jnp.where(y < 255, 1.0, 0.0)
            idxr_v[s] = rw + b * _ROWS
            idxu_v[s] = jnp.maximum(rw - 1, 0) + b * _ROWS
            idxd_v[s] = jnp.minimum(rw + 1, _ROWS - 1) + b * _ROWS
        cpu = pltpu.async_copy(sup_hbm.at[idxu_v], rowsu_v, sem)
        cpc = pltpu.async_copy(sup_hbm.at[idxr_v], rows_v, sem)
        cpd = pltpu.async_copy(sup_hbm.at[idxd_v], rowsd_v, sem)
        cpu.wait()
        cpc.wait()
        cpd.wait()

        for t in range(_CAP // 16):
            wrk_v[pl.ds(t * 16, 16)] = cval_v[pl.ds(t * 16, 16)]

        ninf = jnp.float32(-jnp.inf)

        @plsc.parallel_loop(0, _K, unroll=2)
        def _nms(j):
            jv = spl_i(j)
            uok = plsc.load_gather(upok_v, [jv]) > 0.0
            dok = plsc.load_gather(dnok_v, [jv]) > 0.0
            pend = ninf16
            for c in range(16):
                cols = c * 16 + iot
                colsm = jnp.maximum(cols - 1, 0)
                colsp = jnp.minimum(cols + 1, 255)
                mm = cols >= 1
                mp = cols <= 254
                cc = plsc.load_gather(rows_v, [jv, cols])
                cm_ = plsc.load_gather(rows_v, [jv, colsm])
                cp_ = plsc.load_gather(rows_v, [jv, colsp])
                um = plsc.load_gather(rowsu_v, [jv, colsm])
                uc = plsc.load_gather(rowsu_v, [jv, cols])
                up_ = plsc.load_gather(rowsu_v, [jv, colsp])
                dm = plsc.load_gather(rowsd_v, [jv, colsm])
                dc = plsc.load_gather(rowsd_v, [jv, cols])
                dp_ = plsc.load_gather(rowsd_v, [jv, colsp])
                hc = jnp.maximum(cc, jnp.maximum(
                    jnp.where(mm, cm_, ninf), jnp.where(mp, cp_, ninf)))
                hu = jnp.maximum(uc, jnp.maximum(
                    jnp.where(mm, um, ninf), jnp.where(mp, up_, ninf)))
                hd = jnp.maximum(dc, jnp.maximum(
                    jnp.where(mm, dm, ninf), jnp.where(mp, dp_, ninf)))
                pooled = jnp.maximum(hc, jnp.maximum(
                    jnp.where(uok, hu, ninf), jnp.where(dok, hd, ninf)))
                supc = jnp.where(pooled == cc, cc, 0.0)
                if c > 0:
                    plsc.store_scatter(rows_v, [jv, (c - 1) * 16 + iot], pend)
                pend = supc
                plsc.store_scatter(ccol_v, [spl_i(j * 16 + c)],
                                   spl_f(jnp.max(supc)), mask=lane0)
            plsc.store_scatter(rows_v, [jv, 15 * 16 + iot], pend)

        def el_step(k, carry):
            def sm(t, acc):
                return jnp.maximum(acc, wrk_v[pl.ds(t * 16, 16)])
            m = jnp.max(lax.fori_loop(0, _CAP // 16, sm, ninf16,
                                      unroll=True))

            def kmn(t, acc):
                wv = wrk_v[pl.ds(t * 16, 16)]
                rv = crow_v[pl.ds(t * 16, 16)]
                key = rv * _CAP + (iot + t * 16)
                return jnp.minimum(acc, jnp.where(wv == m, key, _BIGI))
            keymin = jnp.min(lax.fori_loop(
                0, _CAP // 16, kmn, jnp.full((16,), _BIGI_VAL, jnp.int32),
                unroll=True))
            rmin = keymin // _CAP
            j = keymin - rmin * _CAP

            cs = plsc.load_gather(ccol_v, [j * 16 + iot])
            c = jnp.min(jnp.where(cs == m, iot, _BIGI))

            vs = plsc.load_gather(rows_v, [spl_i(j), c * 16 + iot])
            x = c * 16 + jnp.min(jnp.where(vs == m, iot, _BIGI))

            plsc.store_scatter(rows_v, [spl_i(j), spl_i(x)],
                               spl_f(-jnp.inf), mask=lane0)
            ncm = jnp.max(plsc.load_gather(rows_v, [spl_i(j), c * 16 + iot]))
            plsc.store_scatter(ccol_v, [spl_i(j * 16 + c)], spl_f(ncm),
                               mask=lane0)
            nm = jnp.max(plsc.load_gather(ccol_v, [j * 16 + iot]))
            plsc.store_scatter(wrk_v, [spl_i(j)], spl_f(nm), mask=lane0)

            ch = rmin // 256
            y = rmin - ch * 256
            sp = y * 256 + x
            plsc.store_scatter(scv_v, [spl_i(k)], spl_f(m), mask=lane0)
            plsc.store_scatter(idf_v, [spl_i(k)],
                               spl_i(ch).astype(jnp.float32), mask=lane0)
            plsc.store_scatter(xf_v, [spl_i(k)],
                               spl_i(x).astype(jnp.float32), mask=lane0)
            plsc.store_scatter(yf_v, [spl_i(k)],
                               spl_i(y).astype(jnp.float32), mask=lane0)
            plsc.store_scatter(sp_v, [spl_i(k)], spl_i(sp), mask=lane0)
            return carry
        lax.fori_loop(0, _K, el_step, 0)

        for t in range(_CAP // 16):
            base = t * 16 + iot
            pad = base >= _K
            plsc.store_scatter(scv_v, [base], spl_f(-1.0), mask=pad)
            plsc.store_scatter(idf_v, [base], spl_f(-1.0), mask=pad)
            plsc.store_scatter(xf_v, [base], spl_f(0.0), mask=pad)
            plsc.store_scatter(yf_v, [base], spl_f(0.0), mask=pad)
            plsc.store_scatter(sp_v, [base], spl_i(0), mask=pad)

        for t in range(_CAP // 16):
            s = pl.ds(t * 16, 16)
            base = sp_v[s] + b * (2 * _H * _W)
            idx_v[s] = base
            idx2_v[s] = base + _H * _W
        pltpu.async_copy(off_hbm.at[idx_v], ga_v, sem).wait()
        pltpu.async_copy(off_hbm.at[idx2_v], gb_v, sem).wait()
        pltpu.async_copy(wh_hbm.at[idx_v], gc_v, sem).wait()
        pltpu.async_copy(wh_hbm.at[idx2_v], gd_v, sem).wait()

        for t in range(_CAP // 16):
            s = pl.ds(t * 16, 16)
            sc = scv_v[s]
            msk = sc > _THRESH_F
            xs = xf_v[s] + ga_v[s]
            ys = yf_v[s] + gb_v[s]
            hw = gc_v[s] * 0.5
            hh = gd_v[s] * 0.5
            xmin = jnp.where(msk, xs - hw, -1.0) * _SCALE_F
            ymin = jnp.where(msk, ys - hh, -1.0) * _SCALE_F
            xmax = jnp.where(msk, xs + hw, -1.0) * _SCALE_F
            ymax = jnp.where(msk, ys + hh, -1.0) * _SCALE_F
            idf_v[s] = jnp.where(msk, idf_v[s], -1.0)
            scv_v[s] = jnp.where(msk, sc, -1.0)
            rows16 = t * 16 + iot
            plsc.store_scatter(bb_v, [rows16, spl_i(0)], xmin)
            plsc.store_scatter(bb_v, [rows16, spl_i(1)], ymin)
            plsc.store_scatter(bb_v, [rows16, spl_i(2)], xmax)
            plsc.store_scatter(bb_v, [rows16, spl_i(3)], ymax)

        pltpu.sync_copy(idf_v, ids_hbm.at[b])
        pltpu.sync_copy(scv_v, sc_hbm.at[b])
        pltpu.sync_copy(bb_v, bb_hbm.at[b])


def _make_sc_call():
    return functools.partial(
        pl.kernel,
        mesh=plsc.VectorSubcoreMesh(core_axis_name="c", subcore_axis_name="s"),
        compiler_params=pltpu.CompilerParams(needs_layout_passes=False),
        out_type=[
            jax.ShapeDtypeStruct((_B, _CAP), jnp.float32),
            jax.ShapeDtypeStruct((_B, _CAP), jnp.float32),
            jax.ShapeDtypeStruct((_B, _CAP, 4), jnp.float32),
        ],
        scratch_types=[
            pltpu.VMEM((_ROWS,), jnp.float32),
            pltpu.VMEM((_ROWS // 16,), jnp.float32),
            pltpu.VMEM((_NGRP,), jnp.float32),
            pltpu.VMEM((_CAP,), jnp.float32),
            pltpu.VMEM((_CAP,), jnp.int32),
            pltpu.VMEM((_CAPR, _W), jnp.float32),
            pltpu.VMEM((_CAPR, _W), jnp.float32),
            pltpu.VMEM((_CAPR, _W), jnp.float32),
            pltpu.VMEM((_CAP * 16,), jnp.float32),
            pltpu.VMEM((_CAP,), jnp.int32),
            pltpu.VMEM((_CAP,), jnp.int32),
            pltpu.VMEM((_CAPR,), jnp.int32),
            pltpu.VMEM((_CAPR,), jnp.int32),
            pltpu.VMEM((_CAPR,), jnp.int32),
            pltpu.VMEM((_CAPR,), jnp.float32),
            pltpu.VMEM((_CAPR,), jnp.float32),
            pltpu.VMEM((_CAP,), jnp.float32),
            pltpu.VMEM((_CAP,), jnp.float32),
            pltpu.VMEM((_CAP,), jnp.float32),
            pltpu.VMEM((_CAP,), jnp.float32),
            pltpu.VMEM((_CAP,), jnp.int32),
            pltpu.VMEM((_CAP,), jnp.float32),
            pltpu.VMEM((_CAP,), jnp.float32),
            pltpu.VMEM((_CAP,), jnp.float32),
            pltpu.VMEM((_CAP,), jnp.float32),
            pltpu.VMEM((_CAP,), jnp.float32),
            pltpu.VMEM((_CAP, 4), jnp.float32),
            pltpu.SemaphoreType.DMA,
        ],
    )(_sc_body)


def kernel(heatmap, offset, wh):
    (rmax,) = _stage1_nosup(heatmap)
    supf = heatmap.reshape(_B * _ROWS, _W)
    rmax2 = rmax.reshape(_B, _ROWS)
    offf = offset.reshape(-1)
    whf = wh.reshape(-1)
    ids8, sc8, bb8 = _make_sc_call()(rmax2, supf, offf, whf)
    ids = ids8[:, :_K, None]
    scores = sc8[:, :_K, None]
    bboxes = bb8[:, :_K, :]
    return (ids, scores, bboxes)


# speedup vs baseline: 1.1161x; 1.1161x over previous
import functools

import jax
import jax.numpy as jnp
from jax import lax
from jax.experimental import pallas as pl
from jax.experimental.pallas import tpu as pltpu
from jax.experimental.pallas import tpu_sc as plsc

_K = 100
_CAP = 128
_SCALE_F = 4.0
_THRESH_F = 0.01
_CB = 16
_B, _C, _H, _W = 8, 80, 256, 256
_ROWS = _C * _H
_NGRP = _C
_BIGI_VAL = 2 ** 30


def _tc_common(x):
    ninf_row = jnp.full((_CB, 1, _W), -jnp.inf, jnp.float32)
    up = jnp.concatenate([ninf_row, x[:, :-1, :]], axis=1)
    dn = jnp.concatenate([x[:, 1:, :], ninf_row], axis=1)
    v = jnp.maximum(jnp.maximum(up, x), dn)
    ninf_col = jnp.full((_CB, _H, 1), -jnp.inf, jnp.float32)
    lf = jnp.concatenate([ninf_col, v[:, :, :-1]], axis=2)
    rt = jnp.concatenate([v[:, :, 1:], ninf_col], axis=2)
    pooled = jnp.maximum(jnp.maximum(lf, v), rt)
    return jnp.where(pooled == x, x, 0.0)


def _tc_body(hm_ref, sup_ref, rmax_ref):
    sup = _tc_common(hm_ref[0])
    sup_ref[0] = sup
    rmax_ref[0] = jnp.max(sup, axis=-1)


def _tc_body_nosup(hm_ref, rmax_ref):
    sup = _tc_common(hm_ref[0])
    rmax_ref[0] = jnp.max(sup, axis=-1)


_stage1 = pl.pallas_call(
    _tc_body,
    grid=(_B, _C // _CB),
    in_specs=[pl.BlockSpec((1, _CB, _H, _W), lambda b, c: (b, c, 0, 0))],
    out_specs=[
        pl.BlockSpec((1, _CB, _H, _W), lambda b, c: (b, c, 0, 0)),
        pl.BlockSpec((1, _CB, _H), lambda b, c: (b, c, 0)),
    ],
    out_shape=[
        jax.ShapeDtypeStruct((_B, _C, _H, _W), jnp.float32),
        jax.ShapeDtypeStruct((_B, _C, _H), jnp.float32),
    ],
)

_stage1_nosup = pl.pallas_call(
    _tc_body_nosup,
    grid=(_B, _C // _CB),
    in_specs=[pl.BlockSpec((1, _CB, _H, _W), lambda b, c: (b, c, 0, 0))],
    out_specs=[pl.BlockSpec((1, _CB, _H), lambda b, c: (b, c, 0))],
    out_shape=[jax.ShapeDtypeStruct((_B, _C, _H), jnp.float32)],
)


def _sc_body(rmax_hbm, sup_hbm, off_hbm, wh_hbm,
             ids_hbm, sc_hbm, bb_hbm,
             rmax_v, cmax_v, gmax_v, cval_v, crow_v, rows_v, ccol_v,
             idx_v, idx2_v, wrk_v,
             xf_v, yf_v, idf_v, sp_v, scv_v, ga_v, gb_v, gc_v, gd_v, bb_v,
             sem):
    wid = lax.axis_index("s") * 2 + lax.axis_index("c")

    @pl.when(wid < _B)
    def _():
        b = wid
        iot = lax.iota(jnp.int32, 16)
        lane0 = iot == 0
        ninf16 = jnp.full((16,), -jnp.inf, jnp.float32)
        _BIGI = jnp.int32(_BIGI_VAL)

        def spl_i(s):
            return jnp.full((16,), s, jnp.int32)

        def spl_f(s):
            return jnp.full((16,), s, jnp.float32)

        pltpu.sync_copy(rmax_hbm.at[b], rmax_v)

        @plsc.parallel_loop(0, _ROWS // 16, unroll=8)
        def _cpre(c):
            m = jnp.max(plsc.load_gather(rmax_v, [c * 16 + iot]))
            plsc.store_scatter(cmax_v, [spl_i(c)], spl_f(m), mask=lane0)

        @plsc.parallel_loop(0, _NGRP, unroll=8)
        def _gpre(g):
            m = jnp.max(plsc.load_gather(cmax_v, [g * 16 + iot]))
            plsc.store_scatter(gmax_v, [spl_i(g)], spl_f(m), mask=lane0)

        for t in range(_CAP // 16):
            cval_v[pl.ds(t * 16, 16)] = ninf16
            crow_v[pl.ds(t * 16, 16)] = jnp.zeros((16,), jnp.int32)

        def row_step(k, carry):
            def sm(t, acc):
                return jnp.maximum(acc, gmax_v[pl.ds(t * 16, 16)])
            m = jnp.max(lax.fori_loop(0, _NGRP // 16, sm, ninf16,
                                      unroll=True))

            def fg(t, acc):
                v = gmax_v[pl.ds(t * 16, 16)]
                return jnp.minimum(acc, jnp.where(v == m, iot + t * 16, _BIGI))
            g = jnp.min(lax.fori_loop(0, _NGRP // 16, fg,
                                      jnp.full((16,), _BIGI_VAL, jnp.int32),
                                      unroll=True))

            cs = plsc.load_gather(cmax_v, [g * 16 + iot])
            c = g * 16 + jnp.min(jnp.where(cs == m, iot, _BIGI))

            rs = plsc.load_gather(rmax_v, [c * 16 + iot])
            r = c * 16 + jnp.min(jnp.where(rs == m, iot, _BIGI))

            plsc.store_scatter(cval_v, [spl_i(k)], spl_f(m), mask=lane0)
            plsc.store_scatter(crow_v, [spl_i(k)], spl_i(r), mask=lane0)
            plsc.store_scatter(rmax_v, [spl_i(r)], spl_f(-1.0), mask=lane0)

            nc = jnp.max(plsc.load_gather(rmax_v, [c * 16 + iot]))
            plsc.store_scatter(cmax_v, [spl_i(c)], spl_f(nc), mask=lane0)
            ng = jnp.max(plsc.load_gather(cmax_v, [g * 16 + iot]))
            plsc.store_scatter(gmax_v, [spl_i(g)], spl_f(ng), mask=lane0)
            return carry
        lax.fori_loop(0, _K, row_step, 0)

        for t in range(_CAP // 16):
            idx_v[pl.ds(t * 16, 16)] = crow_v[pl.ds(t * 16, 16)] + b * _ROWS
        pltpu.async_copy(sup_hbm.at[idx_v], rows_v, sem).wait()

        for t in range(_CAP // 16):
            wrk_v[pl.ds(t * 16, 16)] = cval_v[pl.ds(t * 16, 16)]

        @plsc.parallel_loop(0, _K * 16, unroll=8)
        def _colpre(i):
            j = i >> 4
            c = i & 15
            m = jnp.max(plsc.load_gather(rows_v, [spl_i(j), c * 16 + iot]))
            plsc.store_scatter(ccol_v, [spl_i(i)], spl_f(m), mask=lane0)

        def el_step(k, carry):
            def sm(t, acc):
                return jnp.maximum(acc, wrk_v[pl.ds(t * 16, 16)])
            m = jnp.max(lax.fori_loop(0, _CAP // 16, sm, ninf16,
                                      unroll=True))

            def kmn(t, acc):
                wv = wrk_v[pl.ds(t * 16, 16)]
                rv = crow_v[pl.ds(t * 16, 16)]
                key = rv * _CAP + (iot + t * 16)
                return jnp.minimum(acc, jnp.where(wv == m, key, _BIGI))
            keymin = jnp.min(lax.fori_loop(
                0, _CAP // 16, kmn, jnp.full((16,), _BIGI_VAL, jnp.int32),
                unroll=True))
            rmin = keymin // _CAP
            j = keymin - rmin * _CAP

            cs = plsc.load_gather(ccol_v, [j * 16 + iot])
            c = jnp.min(jnp.where(cs == m, iot, _BIGI))

            vs = plsc.load_gather(rows_v, [spl_i(j), c * 16 + iot])
            x = c * 16 + jnp.min(jnp.where(vs == m, iot, _BIGI))

            plsc.store_scatter(rows_v, [spl_i(j), spl_i(x)],
                               spl_f(-jnp.inf), mask=lane0)
            ncm = jnp.max(plsc.load_gather(rows_v, [spl_i(j), c * 16 + iot]))
            plsc.store_scatter(ccol_v, [spl_i(j * 16 + c)], spl_f(ncm),
                               mask=lane0)
            nm = jnp.max(plsc.load_gather(ccol_v, [j * 16 + iot]))
            plsc.store_scatter(wrk_v, [spl_i(j)], spl_f(nm), mask=lane0)

            ch = rmin // 256
            y = rmin - ch * 256
            sp = y * 256 + x
            plsc.store_scatter(scv_v, [spl_i(k)], spl_f(m), mask=lane0)
            plsc.store_scatter(idf_v, [spl_i(k)],
                               spl_i(ch).astype(jnp.float32), mask=lane0)
            plsc.store_scatter(xf_v, [spl_i(k)],
                               spl_i(x).astype(jnp.float32), mask=lane0)
            plsc.store_scatter(yf_v, [spl_i(k)],
                               spl_i(y).astype(jnp.float32), mask=lane0)
            plsc.store_scatter(sp_v, [spl_i(k)], spl_i(sp), mask=lane0)
            return carry
        lax.fori_loop(0, _K, el_step, 0)

        for t in range(_CAP // 16):
            base = t * 16 + iot
            pad = base >= _K
            plsc.store_scatter(scv_v, [base], spl_f(-1.0), mask=pad)
            plsc.store_scatter(idf_v, [base], spl_f(-1.0), mask=pad)
            plsc.store_scatter(xf_v, [base], spl_f(0.0), mask=pad)
            plsc.store_scatter(yf_v, [base], spl_f(0.0), mask=pad)
            plsc.store_scatter(sp_v, [base], spl_i(0), mask=pad)

        for t in range(_CAP // 16):
            s = pl.ds(t * 16, 16)
            base = sp_v[s] + b * (2 * _H * _W)
            idx_v[s] = base
            idx2_v[s] = base + _H * _W
        pltpu.async_copy(off_hbm.at[idx_v], ga_v, sem).wait()
        pltpu.async_copy(off_hbm.at[idx2_v], gb_v, sem).wait()
        pltpu.async_copy(wh_hbm.at[idx_v], gc_v, sem).wait()
        pltpu.async_copy(wh_hbm.at[idx2_v], gd_v, sem).wait()

        for t in range(_CAP // 16):
            s = pl.ds(t * 16, 16)
            sc = scv_v[s]
            msk = sc > _THRESH_F
            xs = xf_v[s] + ga_v[s]
            ys = yf_v[s] + gb_v[s]
            hw = gc_v[s] * 0.5
            hh = gd_v[s] * 0.5
            xmin = jnp.where(msk, xs - hw, -1.0) * _SCALE_F
            ymin = jnp.where(msk, ys - hh, -1.0) * _SCALE_F
            xmax = jnp.where(msk, xs + hw, -1.0) * _SCALE_F
            ymax = jnp.where(msk, ys + hh, -1.0) * _SCALE_F
            idf_v[s] = jnp.where(msk, idf_v[s], -1.0)
            scv_v[s] = jnp.where(msk, sc, -1.0)
            rows16 = t * 16 + iot
            plsc.store_scatter(bb_v, [rows16, spl_i(0)], xmin)
            plsc.store_scatter(bb_v, [rows16, spl_i(1)], ymin)
            plsc.store_scatter(bb_v, [rows16, spl_i(2)], xmax)
            plsc.store_scatter(bb_v, [rows16, spl_i(3)], ymax)

        pltpu.sync_copy(idf_v, ids_hbm.at[b])
        pltpu.sync_copy(scv_v, sc_hbm.at[b])
        pltpu.sync_copy(bb_v, bb_hbm.at[b])


def _make_sc_call():
    return functools.partial(
        pl.kernel,
        mesh=plsc.VectorSubcoreMesh(core_axis_name="c", subcore_axis_name="s"),
        compiler_params=pltpu.CompilerParams(needs_layout_passes=False),
        out_type=[
            jax.ShapeDtypeStruct((_B, _CAP), jnp.float32),
            jax.ShapeDtypeStruct((_B, _CAP), jnp.float32),
            jax.ShapeDtypeStruct((_B, _CAP, 4), jnp.float32),
        ],
        scratch_types=[
            pltpu.VMEM((_ROWS,), jnp.float32),
            pltpu.VMEM((_ROWS // 16,), jnp.float32),
            pltpu.VMEM((_NGRP,), jnp.float32),
            pltpu.VMEM((_CAP,), jnp.float32),
            pltpu.VMEM((_CAP,), jnp.int32),
            pltpu.VMEM((_CAP, _W), jnp.float32),
            pltpu.VMEM((_CAP * 16,), jnp.float32),
            pltpu.VMEM((_CAP,), jnp.int32),
            pltpu.VMEM((_CAP,), jnp.int32),
            pltpu.VMEM((_CAP,), jnp.float32),
            pltpu.VMEM((_CAP,), jnp.float32),
            pltpu.VMEM((_CAP,), jnp.float32),
            pltpu.VMEM((_CAP,), jnp.float32),
            pltpu.VMEM((_CAP,), jnp.int32),
            pltpu.VMEM((_CAP,), jnp.float32),
            pltpu.VMEM((_CAP,), jnp.float32),
            pltpu.VMEM((_CAP,), jnp.float32),
            pltpu.VMEM((_CAP,), jnp.float32),
            pltpu.VMEM((_CAP,), jnp.float32),
            pltpu.VMEM((_CAP, 4), jnp.float32),
            pltpu.SemaphoreType.DMA,
        ],
    )(_sc_body)


def kernel(heatmap, offset, wh):
    sup, rmax = _stage1(heatmap)
    supf = sup.reshape(_B * _ROWS, _W)
    rmax2 = rmax.reshape(_B, _ROWS)
    offf = offset.reshape(-1)
    whf = wh.reshape(-1)
    ids8, sc8, bb8 = _make_sc_call()(rmax2, supf, offf, whf)
    ids = ids8[:, :_K, None]
    scores = sc8[:, :_K, None]
    bboxes = bb8[:, :_K, :]
    return (ids, scores, bboxes)


# speedup vs baseline: 1.1178x; 1.0015x over previous
import functools

import jax
import jax.numpy as jnp
from jax import lax
from jax.experimental import pallas as pl
from jax.experimental.pallas import tpu as pltpu
from jax.experimental.pallas import tpu_sc as plsc

_K = 100
_CAP = 128
_SCALE_F = 4.0
_THRESH_F = 0.01
_CB = 16
_B, _C, _H, _W = 8, 80, 256, 256
_ROWS = _C * _H
_NGRP = _C
_BIGI_VAL = 2 ** 30


def _tc_common(x):
    ninf_row = jnp.full((_CB, 1, _W), -jnp.inf, jnp.float32)
    up = jnp.concatenate([ninf_row, x[:, :-1, :]], axis=1)
    dn = jnp.concatenate([x[:, 1:, :], ninf_row], axis=1)
    v = jnp.maximum(jnp.maximum(up, x), dn)
    ninf_col = jnp.full((_CB, _H, 1), -jnp.inf, jnp.float32)
    lf = jnp.concatenate([ninf_col, v[:, :, :-1]], axis=2)
    rt = jnp.concatenate([v[:, :, 1:], ninf_col], axis=2)
    pooled = jnp.maximum(jnp.maximum(lf, v), rt)
    return jnp.where(pooled == x, x, 0.0)


def _tc_body(hm_ref, sup_ref, rmax_ref):
    sup = _tc_common(hm_ref[0])
    sup_ref[0] = sup
    rmax_ref[0] = jnp.max(sup, axis=-1)


_stage1 = pl.pallas_call(
    _tc_body,
    grid=(_B, _C // _CB),
    in_specs=[pl.BlockSpec((1, _CB, _H, _W), lambda b, c: (b, c, 0, 0))],
    out_specs=[
        pl.BlockSpec((1, _CB, _H, _W), lambda b, c: (b, c, 0, 0)),
        pl.BlockSpec((1, _CB, _H), lambda b, c: (b, c, 0)),
    ],
    out_shape=[
        jax.ShapeDtypeStruct((_B, _C, _H, _W), jnp.float32),
        jax.ShapeDtypeStruct((_B, _C, _H), jnp.float32),
    ],
)


def _sc_body(rmax_hbm, sup_hbm, off_hbm, wh_hbm,
             ids_hbm, sc_hbm, bb_hbm,
             rmax_v, cmax_v, gmax_v, cval_v, crow_v, rows_v, ccol_v,
             idx_v, idx2_v, wrk_v,
             xf_v, yf_v, idf_v, sp_v, scv_v, ga_v, gb_v, gc_v, gd_v, bb_v,
             sem):
    wid = lax.axis_index("s") * 2 + lax.axis_index("c")

    @pl.when(wid < _B)
    def _():
        b = wid
        iot = lax.iota(jnp.int32, 16)
        lane0 = iot == 0
        ninf16 = jnp.full((16,), -jnp.inf, jnp.float32)
        _BIGI = jnp.int32(_BIGI_VAL)

        def spl_i(s):
            return jnp.full((16,), s, jnp.int32)

        def spl_f(s):
            return jnp.full((16,), s, jnp.float32)

        pltpu.sync_copy(rmax_hbm.at[b], rmax_v)

        @plsc.parallel_loop(0, _ROWS // 16, unroll=8)
        def _cpre(c):
            m = jnp.max(plsc.load_gather(rmax_v, [c * 16 + iot]))
            plsc.store_scatter(cmax_v, [spl_i(c)], spl_f(m), mask=lane0)

        @plsc.parallel_loop(0, _NGRP, unroll=8)
        def _gpre(g):
            m = jnp.max(plsc.load_gather(cmax_v, [g * 16 + iot]))
            plsc.store_scatter(gmax_v, [spl_i(g)], spl_f(m), mask=lane0)

        for t in range(_CAP // 16):
            cval_v[pl.ds(t * 16, 16)] = ninf16
            crow_v[pl.ds(t * 16, 16)] = jnp.zeros((16,), jnp.int32)

        def row_step(k, carry):
            def smfg(t, carry):
                mx, gid = carry
                v = gmax_v[pl.ds(t * 16, 16)]
                gid = jnp.where(v > mx, iot + t * 16, gid)
                return (jnp.maximum(mx, v), gid)
            mx, gid = lax.fori_loop(
                0, _NGRP // 16, smfg,
                (ninf16, jnp.full((16,), _BIGI_VAL, jnp.int32)), unroll=True)
            m = jnp.max(mx)
            g = jnp.min(jnp.where(mx == m, gid, _BIGI))

            cs = plsc.load_gather(cmax_v, [g * 16 + iot])
            c = g * 16 + jnp.min(jnp.where(cs == m, iot, _BIGI))

            rs = plsc.load_gather(rmax_v, [c * 16 + iot])
            r = c * 16 + jnp.min(jnp.where(rs == m, iot, _BIGI))

            plsc.store_scatter(cval_v, [spl_i(k)], spl_f(m), mask=lane0)
            plsc.store_scatter(crow_v, [spl_i(k)], spl_i(r), mask=lane0)
            plsc.store_scatter(rmax_v, [spl_i(r)], spl_f(-1.0), mask=lane0)

            nc = jnp.max(plsc.load_gather(rmax_v, [c * 16 + iot]))
            plsc.store_scatter(cmax_v, [spl_i(c)], spl_f(nc), mask=lane0)
            ng = jnp.max(plsc.load_gather(cmax_v, [g * 16 + iot]))
            plsc.store_scatter(gmax_v, [spl_i(g)], spl_f(ng), mask=lane0)
            return carry
        lax.fori_loop(0, _K, row_step, 0)

        for t in range(_CAP // 16):
            idx_v[pl.ds(t * 16, 16)] = crow_v[pl.ds(t * 16, 16)] + b * _ROWS
        pltpu.async_copy(sup_hbm.at[idx_v], rows_v, sem).wait()

        for t in range(_CAP // 16):
            wrk_v[pl.ds(t * 16, 16)] = cval_v[pl.ds(t * 16, 16)]

        @plsc.parallel_loop(0, _K * 16, unroll=8)
        def _colpre(i):
            j = i >> 4
            c = i & 15
            m = jnp.max(plsc.load_gather(rows_v, [spl_i(j), c * 16 + iot]))
            plsc.store_scatter(ccol_v, [spl_i(i)], spl_f(m), mask=lane0)

        def el_step(k, carry):
            def smkm(t, carry):
                mx, key = carry
                wv = wrk_v[pl.ds(t * 16, 16)]
                rv = crow_v[pl.ds(t * 16, 16)]
                nkey = rv * _CAP + (iot + t * 16)
                key = jnp.where(wv > mx, nkey,
                                jnp.where(wv == mx,
                                          jnp.minimum(key, nkey), key))
                return (jnp.maximum(mx, wv), key)
            mx, key = lax.fori_loop(
                0, _CAP // 16, smkm,
                (ninf16, jnp.full((16,), _BIGI_VAL, jnp.int32)), unroll=True)
            m = jnp.max(mx)
            keymin = jnp.min(jnp.where(mx == m, key, _BIGI))
            rmin = keymin // _CAP
            j = keymin - rmin * _CAP

            cs = plsc.load_gather(ccol_v, [j * 16 + iot])
            c = jnp.min(jnp.where(cs == m, iot, _BIGI))

            vs = plsc.load_gather(rows_v, [spl_i(j), c * 16 + iot])
            x = c * 16 + jnp.min(jnp.where(vs == m, iot, _BIGI))

            plsc.store_scatter(rows_v, [spl_i(j), spl_i(x)],
                               spl_f(-jnp.inf), mask=lane0)
            ncm = jnp.max(plsc.load_gather(rows_v, [spl_i(j), c * 16 + iot]))
            plsc.store_scatter(ccol_v, [spl_i(j * 16 + c)], spl_f(ncm),
                               mask=lane0)
            nm = jnp.max(plsc.load_gather(ccol_v, [j * 16 + iot]))
            plsc.store_scatter(wrk_v, [spl_i(j)], spl_f(nm), mask=lane0)

            ch = rmin // 256
            y = rmin - ch * 256
            sp = y * 256 + x
            plsc.store_scatter(scv_v, [spl_i(k)], spl_f(m), mask=lane0)
            plsc.store_scatter(idf_v, [spl_i(k)],
                               spl_i(ch).astype(jnp.float32), mask=lane0)
            plsc.store_scatter(xf_v, [spl_i(k)],
                               spl_i(x).astype(jnp.float32), mask=lane0)
            plsc.store_scatter(yf_v, [spl_i(k)],
                               spl_i(y).astype(jnp.float32), mask=lane0)
            plsc.store_scatter(sp_v, [spl_i(k)], spl_i(sp), mask=lane0)
            return carry
        lax.fori_loop(0, _K, el_step, 0)

        for t in range(_CAP // 16):
            base = t * 16 + iot
            pad = base >= _K
            plsc.store_scatter(scv_v, [base], spl_f(-1.0), mask=pad)
            plsc.store_scatter(idf_v, [base], spl_f(-1.0), mask=pad)
            plsc.store_scatter(xf_v, [base], spl_f(0.0), mask=pad)
            plsc.store_scatter(yf_v, [base], spl_f(0.0), mask=pad)
            plsc.store_scatter(sp_v, [base], spl_i(0), mask=pad)

        for t in range(_CAP // 16):
            s = pl.ds(t * 16, 16)
            base = sp_v[s] + b * (2 * _H * _W)
            idx_v[s] = base
            idx2_v[s] = base + _H * _W
        pltpu.async_copy(off_hbm.at[idx_v], ga_v, sem).wait()
        pltpu.async_copy(off_hbm.at[idx2_v], gb_v, sem).wait()
        pltpu.async_copy(wh_hbm.at[idx_v], gc_v, sem).wait()
        pltpu.async_copy(wh_hbm.at[idx2_v], gd_v, sem).wait()

        for t in range(_CAP // 16):
            s = pl.ds(t * 16, 16)
            sc = scv_v[s]
            msk = sc > _THRESH_F
            xs = xf_v[s] + ga_v[s]
            ys = yf_v[s] + gb_v[s]
            hw = gc_v[s] * 0.5
            hh = gd_v[s] * 0.5
            xmin = jnp.where(msk, xs - hw, -1.0) * _SCALE_F
            ymin = jnp.where(msk, ys - hh, -1.0) * _SCALE_F
            xmax = jnp.where(msk, xs + hw, -1.0) * _SCALE_F
            ymax = jnp.where(msk, ys + hh, -1.0) * _SCALE_F
            idf_v[s] = jnp.where(msk, idf_v[s], -1.0)
            scv_v[s] = jnp.where(msk, sc, -1.0)
            rows16 = t * 16 + iot
            plsc.store_scatter(bb_v, [rows16, spl_i(0)], xmin)
            plsc.store_scatter(bb_v, [rows16, spl_i(1)], ymin)
            plsc.store_scatter(bb_v, [rows16, spl_i(2)], xmax)
            plsc.store_scatter(bb_v, [rows16, spl_i(3)], ymax)

        pltpu.sync_copy(idf_v, ids_hbm.at[b])
        pltpu.sync_copy(scv_v, sc_hbm.at[b])
        pltpu.sync_copy(bb_v, bb_hbm.at[b])


def _make_sc_call():
    return functools.partial(
        pl.kernel,
        mesh=plsc.VectorSubcoreMesh(core_axis_name="c", subcore_axis_name="s"),
        compiler_params=pltpu.CompilerParams(needs_layout_passes=False),
        out_type=[
            jax.ShapeDtypeStruct((_B, _CAP), jnp.float32),
            jax.ShapeDtypeStruct((_B, _CAP), jnp.float32),
            jax.ShapeDtypeStruct((_B, _CAP, 4), jnp.float32),
        ],
        scratch_types=[
            pltpu.VMEM((_ROWS,), jnp.float32),
            pltpu.VMEM((_ROWS // 16,), jnp.float32),
            pltpu.VMEM((_NGRP,), jnp.float32),
            pltpu.VMEM((_CAP,), jnp.float32),
            pltpu.VMEM((_CAP,), jnp.int32),
            pltpu.VMEM((_CAP, _W), jnp.float32),
            pltpu.VMEM((_CAP * 16,), jnp.float32),
            pltpu.VMEM((_CAP,), jnp.int32),
            pltpu.VMEM((_CAP,), jnp.int32),
            pltpu.VMEM((_CAP,), jnp.float32),
            pltpu.VMEM((_CAP,), jnp.float32),
            pltpu.VMEM((_CAP,), jnp.float32),
            pltpu.VMEM((_CAP,), jnp.float32),
            pltpu.VMEM((_CAP,), jnp.int32),
            pltpu.VMEM((_CAP,), jnp.float32),
            pltpu.VMEM((_CAP,), jnp.float32),
            pltpu.VMEM((_CAP,), jnp.float32),
            pltpu.VMEM((_CAP,), jnp.float32),
            pltpu.VMEM((_CAP,), jnp.float32),
            pltpu.VMEM((_CAP, 4), jnp.float32),
            pltpu.SemaphoreType.DMA,
        ],
    )(_sc_body)


def kernel(heatmap, offset, wh):
    sup, rmax = _stage1(heatmap)
    supf = sup.reshape(_B * _ROWS, _W)
    rmax2 = rmax.reshape(_B, _ROWS)
    offf = offset.reshape(-1)
    whf = wh.reshape(-1)
    ids8, sc8, bb8 = _make_sc_call()(rmax2, supf, offf, whf)
    ids = ids8[:, :_K, None]
    scores = sc8[:, :_K, None]
    bboxes = bb8[:, :_K, :]
    return (ids, scores, bboxes)
